# baseline (speedup 1.0000x reference)
import functools

import jax
import jax.numpy as jnp
from jax import lax
from jax.experimental import pallas as pl
from jax.experimental.pallas import tpu as pltpu
from jax.experimental.pallas import tpu_sc as plsc

_B = 128
_V = 100000
_NC = 2
_NS = 16
_NW = _NC * _NS
_RPW = _B // _NW
_L = 16
_C = 10000
_NCHUNK = _V // _C
_U = 25
_ITERS = (_C // _L) // _U

_NEG_INF = float("-inf")


def _sc_body(logits, act, g_out, m_out, s_out, i_out,
             buf, act_v, gst, mst, sst, ist):
    wid = lax.axis_index("s") * _NC + lax.axis_index("c")
    base = wid * _RPW
    lane = lax.iota(jnp.int32, _L)
    pltpu.sync_copy(act, act_v)
    avec = plsc.load_gather(act_v, [base + jnp.minimum(lane, _RPW - 1)])

    g_acc = jnp.zeros((_L,), jnp.float32)
    for j in range(_RPW):
        row = base + j

        def chunk_body(c, car, j=j):
            m0, ix0, s0, g0 = car
            off = c * _C
            pltpu.sync_copy(logits.at[pl.ds(row * _V + off, _C)], buf)

            def p1(i, car1):
                m, ix = car1
                for u in range(_U):
                    o = i * (_U * _L) + u * _L
                    x = buf[pl.ds(o, _L)]
                    gt = x > m
                    m = jnp.where(gt, x, m)
                    ix = jnp.where(gt, off + o + lane, ix)
                return m, ix

            m1, ix1 = lax.fori_loop(0, _ITERS, p1, (m0, ix0))

            loc = avec - off
            inb = (loc >= 0) & (loc < _C) & (lane == j)
            gath = plsc.load_gather(buf, [jnp.clip(loc, 0, _C - 1)])
            g1 = jnp.where(inb, gath, g0)

            s1 = jnp.where(m0 > _NEG_INF, s0 * jnp.exp(m0 - m1), 0.0)

            def p2(i, s):
                for u in range(_U):
                    o = i * (_U * _L) + u * _L
                    s = s + jnp.exp(buf[pl.ds(o, _L)] - m1)
                return s

            s2 = lax.fori_loop(0, _ITERS, p2, s1)
            return m1, ix1, s2, g1

        init = (jnp.full((_L,), _NEG_INF, jnp.float32),
                jnp.zeros((_L,), jnp.int32),
                jnp.zeros((_L,), jnp.float32),
                g_acc)
        m, ix, s, g_acc = lax.fori_loop(0, _NCHUNK, chunk_body, init)
        mst[j, :] = m
        ist[j, :] = ix
        sst[j, :] = s

    gst[...] = g_acc
    pltpu.sync_copy(gst, g_out.at[wid])
    pltpu.sync_copy(mst, m_out.at[wid])
    pltpu.sync_copy(sst, s_out.at[wid])
    pltpu.sync_copy(ist, i_out.at[wid])


_sc_part = functools.partial(
    pl.kernel,
    out_type=(
        jax.ShapeDtypeStruct((_NW, _L), jnp.float32),
        jax.ShapeDtypeStruct((_NW, _RPW, _L), jnp.float32),
        jax.ShapeDtypeStruct((_NW, _RPW, _L), jnp.float32),
        jax.ShapeDtypeStruct((_NW, _RPW, _L), jnp.int32),
    ),
    mesh=plsc.VectorSubcoreMesh(
        core_axis_name="c", subcore_axis_name="s",
        num_cores=_NC, num_subcores=_NS),
    compiler_params=pltpu.CompilerParams(
        use_tc_tiling_on_sc=False, needs_layout_passes=False),
    scratch_types=[
        pltpu.VMEM((_C,), jnp.float32),
        pltpu.VMEM((_B,), jnp.int32),
        pltpu.VMEM((_L,), jnp.float32),
        pltpu.VMEM((_RPW, _L), jnp.float32),
        pltpu.VMEM((_RPW, _L), jnp.float32),
        pltpu.VMEM((_RPW, _L), jnp.int32),
    ],
)(_sc_body)


def _tc_merge(m_ref, s_ref, i_ref, g_ref, lp_ref, md_ref):
    m = m_ref[...]
    mx = jnp.max(m, axis=1, keepdims=True)
    stot = jnp.sum(s_ref[...] * jnp.exp(m - mx), axis=1, keepdims=True)
    lp_ref[...] = g_ref[...] - (mx + jnp.log(stot))
    big = jnp.where(m == mx, i_ref[...], jnp.int32(2**31 - 1))
    md_ref[...] = jnp.min(big, axis=1, keepdims=True)


def kernel(logits, actions):
    act = actions.reshape(_B).astype(jnp.int32)
    g_out, m_out, s_out, i_out = _sc_part(logits.reshape(_B * _V), act)
    lp, md = pl.pallas_call(
        _tc_merge,
        out_shape=(jax.ShapeDtypeStruct((_B, 1), jnp.float32),
                   jax.ShapeDtypeStruct((_B, 1), jnp.int32)),
    )(m_out.reshape(_B, _L), s_out.reshape(_B, _L),
      i_out.reshape(_B, _L), g_out[:, :_RPW].reshape(_B, 1))
    return lp, md


# speedup vs baseline: 1.6410x; 1.6410x over previous
import functools

import jax
import jax.numpy as jnp
import numpy as np
from jax import lax
from jax.experimental import pallas as pl
from jax.experimental.pallas import tpu as pltpu
from jax.experimental.pallas import tpu_sc as plsc

_B = 128
_V = 100000
_L = 16
_G = 16
_R8 = 8
_CW = 2944
_BOUND = 50048
_TW = 2816
_TCW = _V - _BOUND - 16 * _CW - _TW
_NV = _CW // _L
_NVT = _TW // _L
_U1 = 23
_UT = 22

_NEG_INF = float("-inf")
_I32_BIG = np.int32(2**31 - 1)


def _sc_body(logits, act, g_out, m_out, s_out, i_out,
             buf_a, buf_b, buf_t, act_v, gst, mst, sst, ist,
             sem_a, sem_b, sem_t):
    h = lax.axis_index("c")
    g = lax.axis_index("s")
    cb = h * _BOUND
    lane = lax.iota(jnp.int32, _L)
    neg = jnp.full((_L,), _NEG_INF, jnp.float32)
    zf = jnp.zeros((_L,), jnp.float32)
    zi = jnp.zeros((_L,), jnp.int32)

    pltpu.sync_copy(act, act_v)
    for r in range(_R8):
        mst[r, :] = neg
        ist[r, :] = zi
        sst[r, :] = zf
    gst[...] = zf

    def dma(c, buf, sem):
        pltpu.async_copy(
            logits.at[pl.ds(8 * g, _R8), pl.ds(cb + c * _CW, _CW)], buf, sem)

    def wait(buf, sem):
        pltpu.make_async_copy(
            logits.at[pl.ds(0, _R8), pl.ds(0, _CW)], buf, sem).wait()

    def process(buf, off, nv, un):
        def row_body(r, _):
            mv = mst[r, :]
            iv = ist[r, :]
            sv = sst[r, :]

            def p1(i, car):
                m, ix = car
                for u in range(un):
                    o = (i * un + u) * _L
                    x = buf[r, pl.ds(o, _L)]
                    nm = jnp.maximum(m, x)
                    gt = x > m
                    ix = jnp.where(gt, off + o + lane, ix)
                    m = nm
                return m, ix

            m1, ix1 = lax.fori_loop(0, nv // un, p1, (mv, iv))

            s1 = jnp.where(mv > _NEG_INF, sv * jnp.exp(mv - m1), 0.0)

            def p2(i, s):
                for u in range(un):
                    o = (i * un + u) * _L
                    s = s + jnp.exp(buf[r, pl.ds(o, _L)] - m1)
                return s

            s2 = lax.fori_loop(0, nv // un, p2, s1)

            mst[r, :] = m1
            ist[r, :] = ix1
            sst[r, :] = s2

            ab = plsc.load_gather(act_v, [zi + (_R8 * g + r)])
            loc = ab - off
            inb = (loc >= 0) & (loc < nv * _L) & (lane == r)
            gath = plsc.load_gather(
                buf, [zi + r, jnp.clip(loc, 0, nv * _L - 1)])
            gst[...] = jnp.where(inb, gath, gst[...])
            return 0

        lax.fori_loop(0, _R8, row_body, 0)

    dma(0, buf_a, sem_a)
    dma(1, buf_b, sem_b)

    @pl.loop(0, 16, step=2)
    def main_loop(c):
        wait(buf_a, sem_a)
        process(buf_a, cb + c * _CW, _NV, _U1)

        @pl.when(c < 14)
        def _():
            dma(c + 2, buf_a, sem_a)

        @pl.when(c == 14)
        def _():
            @pl.when(h == 0)
            def _():
                dma(16, buf_a, sem_a)

            @pl.when(h == 1)
            def _():
                pltpu.async_copy(
                    logits.at[pl.ds(8 * g, _R8),
                              pl.ds(_BOUND + 16 * _CW, _TW)],
                    buf_t, sem_t)

        wait(buf_b, sem_b)
        process(buf_b, cb + (c + 1) * _CW, _NV, _U1)

        @pl.when(c < 13)
        def _():
            dma(c + 3, buf_b, sem_b)

    @pl.when(h == 0)
    def _():
        wait(buf_a, sem_a)
        process(buf_a, 16 * _CW, _NV, _U1)

    @pl.when(h == 1)
    def _():
        pltpu.make_async_copy(
            logits.at[pl.ds(0, _R8), pl.ds(0, _TW)], buf_t, sem_t).wait()
        process(buf_t, _BOUND + 16 * _CW, _NVT, _UT)

    pltpu.sync_copy(gst, g_out.at[h, g])
    pltpu.sync_copy(mst, m_out.at[h, g])
    pltpu.sync_copy(sst, s_out.at[h, g])
    pltpu.sync_copy(ist, i_out.at[h, g])


_sc_part = functools.partial(
    pl.kernel,
    out_type=(
        jax.ShapeDtypeStruct((2, _G, _L), jnp.float32),
        jax.ShapeDtypeStruct((2, _G, _R8, _L), jnp.float32),
        jax.ShapeDtypeStruct((2, _G, _R8, _L), jnp.float32),
        jax.ShapeDtypeStruct((2, _G, _R8, _L), jnp.int32),
    ),
    mesh=plsc.VectorSubcoreMesh(
        core_axis_name="c", subcore_axis_name="s",
        num_cores=2, num_subcores=16),
    compiler_params=pltpu.CompilerParams(needs_layout_passes=False),
    scratch_types=[
        pltpu.VMEM((_R8, _CW), jnp.float32),
        pltpu.VMEM((_R8, _CW), jnp.float32),
        pltpu.VMEM((_R8, _TW), jnp.float32),
        pltpu.VMEM((_B,), jnp.int32),
        pltpu.VMEM((_L,), jnp.float32),
        pltpu.VMEM((_R8, _L), jnp.float32),
        pltpu.VMEM((_R8, _L), jnp.float32),
        pltpu.VMEM((_R8, _L), jnp.int32),
        pltpu.SemaphoreType.DMA,
        pltpu.SemaphoreType.DMA,
        pltpu.SemaphoreType.DMA,
    ],
)(_sc_body)


def _tc_merge(ma_ref, mb_ref, sa_ref, sb_ref, ia_ref, ib_ref,
              ga_ref, gb_ref, tail_ref, act_ref, lp_ref, md_ref):
    ma = ma_ref[...]
    mb = mb_ref[...]
    t = tail_ref[...]
    cols = jnp.int32(_V - _TCW) + jax.lax.broadcasted_iota(
        jnp.int32, (_B, _TCW), 1)
    mx = jnp.maximum(jnp.max(ma, axis=1, keepdims=True),
                     jnp.max(mb, axis=1, keepdims=True))
    mx = jnp.maximum(mx, jnp.max(t, axis=1, keepdims=True))
    stot = (jnp.sum(sa_ref[...] * jnp.exp(ma - mx), axis=1, keepdims=True)
            + jnp.sum(sb_ref[...] * jnp.exp(mb - mx), axis=1, keepdims=True)
            + jnp.sum(jnp.exp(t - mx), axis=1, keepdims=True))
    gt = jnp.sum(jnp.where(cols == act_ref[...], t, 0.0),
                 axis=1, keepdims=True)
    lp_ref[...] = (ga_ref[...] + gb_ref[...] + gt) - (mx + jnp.log(stot))
    ca = jnp.min(jnp.where(ma == mx, ia_ref[...], _I32_BIG),
                 axis=1, keepdims=True)
    cb = jnp.min(jnp.where(mb == mx, ib_ref[...], _I32_BIG),
                 axis=1, keepdims=True)
    ct = jnp.min(jnp.where(t == mx, cols, _I32_BIG),
                 axis=1, keepdims=True)
    md_ref[...] = jnp.minimum(jnp.minimum(ca, cb), ct)


def kernel(logits, actions):
    act = actions.reshape(_B).astype(jnp.int32)
    g_out, m_out, s_out, i_out = _sc_part(logits, act)
    lp, md = pl.pallas_call(
        _tc_merge,
        out_shape=(jax.ShapeDtypeStruct((_B, 1), jnp.float32),
                   jax.ShapeDtypeStruct((_B, 1), jnp.int32)),
    )(m_out[0].reshape(_B, _L), m_out[1].reshape(_B, _L),
      s_out[0].reshape(_B, _L), s_out[1].reshape(_B, _L),
      i_out[0].reshape(_B, _L), i_out[1].reshape(_B, _L),
      g_out[0, :, :_R8].reshape(_B, 1), g_out[1, :, :_R8].reshape(_B, 1),
      logits[:, _V - _TCW:], act[:, None])
    return lp, md


# speedup vs baseline: 2.9524x; 1.7991x over previous
import functools

import jax
import jax.numpy as jnp
import numpy as np
from jax import lax
from jax.experimental import pallas as pl
from jax.experimental.pallas import tpu as pltpu
from jax.experimental.pallas import tpu_sc as plsc

_B = 128
_V = 100000
_L = 16
_NW = 32
_NBG = _B // _L
_VW = 3128
_W = 184
_TWR = _V - 31 * _VW - 16 * _W
_U1 = 23
_UT = 11

_NEG_INF = float("-inf")
_I32_BIG = np.int32(2**31 - 1)


def _sc_body(lt, act, g_out, m_out, s_out, i_out,
             buf_a, buf_b, buf_t, act_v, gst, mst, sst, ist,
             sem_a, sem_b, sem_t):
    w = lax.axis_index("s") * 2 + lax.axis_index("c")
    vb = w * _VW
    lane = lax.iota(jnp.int32, _L)
    neg = jnp.full((_L,), _NEG_INF, jnp.float32)
    zf = jnp.zeros((_L,), jnp.float32)
    zi = jnp.zeros((_L,), jnp.int32)

    pltpu.sync_copy(act, act_v)
    for bg in range(_NBG):
        mst[bg, :] = neg
        ist[bg, :] = zi
        sst[bg, :] = zf
        gst[bg, :] = zf

    def dma(c, buf, sem):
        pltpu.async_copy(lt.at[pl.ds(vb + c * _W, _W), :], buf, sem)

    def wait(buf, sem):
        pltpu.make_async_copy(lt.at[pl.ds(0, _W), :], buf, sem).wait()

    def process(buf, off, nv, un):
        def bg_body(bg, _):
            mv = mst[bg, :]
            iv = ist[bg, :]
            sv = sst[bg, :]

            def p1(i, car):
                m, ix = car
                for u in range(un):
                    o = i * un + u
                    x = buf[o, pl.ds(bg * _L, _L)]
                    nm = jnp.maximum(m, x)
                    ix = jnp.where(x > m, zi + (off + o), ix)
                    m = nm
                return m, ix

            m1, ix1 = lax.fori_loop(0, nv // un, p1, (mv, iv))

            s1 = jnp.where(mv > _NEG_INF, sv * jnp.exp(mv - m1), 0.0)

            def p2(i, s):
                for u in range(un):
                    o = i * un + u
                    s = s + jnp.exp(buf[o, pl.ds(bg * _L, _L)] - m1)
                return s

            s2 = lax.fori_loop(0, nv // un, p2, s1)

            mst[bg, :] = m1
            ist[bg, :] = ix1
            sst[bg, :] = s2

            av = act_v[pl.ds(bg * _L, _L)]
            loc = av - off
            inb = (loc >= 0) & (loc < nv)
            gath = plsc.load_gather(
                buf, [jnp.clip(loc, 0, nv - 1), bg * _L + lane])
            gst[bg, :] = jnp.where(inb, gath, gst[bg, :])
            return 0

        lax.fori_loop(0, _NBG, bg_body, 0)

    dma(0, buf_a, sem_a)
    dma(1, buf_b, sem_b)

    @pl.loop(0, 16, step=2)
    def main_loop(c):
        wait(buf_a, sem_a)
        process(buf_a, vb + c * _W, _W, _U1)

        @pl.when(c < 14)
        def _():
            dma(c + 2, buf_a, sem_a)

        @pl.when(c == 14)
        def _():
            @pl.when(w < 31)
            def _():
                dma(16, buf_a, sem_a)

            @pl.when(w == 31)
            def _():
                pltpu.async_copy(
                    lt.at[pl.ds(vb + 16 * _W, _TWR), :], buf_t, sem_t)

        wait(buf_b, sem_b)
        process(buf_b, vb + (c + 1) * _W, _W, _U1)

        @pl.when(c < 13)
        def _():
            dma(c + 3, buf_b, sem_b)

    @pl.when(w < 31)
    def _():
        wait(buf_a, sem_a)
        process(buf_a, vb + 16 * _W, _W, _U1)

    @pl.when(w == 31)
    def _():
        pltpu.make_async_copy(
            lt.at[pl.ds(0, _TWR), :], buf_t, sem_t).wait()
        process(buf_t, vb + 16 * _W, _TWR, _UT)

    pltpu.sync_copy(gst, g_out.at[w])
    pltpu.sync_copy(mst, m_out.at[w])
    pltpu.sync_copy(sst, s_out.at[w])
    pltpu.sync_copy(ist, i_out.at[w])


_sc_part = functools.partial(
    pl.kernel,
    out_type=(
        jax.ShapeDtypeStruct((_NW, _NBG, _L), jnp.float32),
        jax.ShapeDtypeStruct((_NW, _NBG, _L), jnp.float32),
        jax.ShapeDtypeStruct((_NW, _NBG, _L), jnp.float32),
        jax.ShapeDtypeStruct((_NW, _NBG, _L), jnp.int32),
    ),
    mesh=plsc.VectorSubcoreMesh(
        core_axis_name="c", subcore_axis_name="s",
        num_cores=2, num_subcores=16),
    compiler_params=pltpu.CompilerParams(needs_layout_passes=False),
    scratch_types=[
        pltpu.VMEM((_W, _B), jnp.float32),
        pltpu.VMEM((_W, _B), jnp.float32),
        pltpu.VMEM((_TWR, _B), jnp.float32),
        pltpu.VMEM((_B,), jnp.int32),
        pltpu.VMEM((_NBG, _L), jnp.float32),
        pltpu.VMEM((_NBG, _L), jnp.float32),
        pltpu.VMEM((_NBG, _L), jnp.float32),
        pltpu.VMEM((_NBG, _L), jnp.int32),
        pltpu.SemaphoreType.DMA,
        pltpu.SemaphoreType.DMA,
        pltpu.SemaphoreType.DMA,
    ],
)(_sc_body)


def _tc_merge(m_ref, s_ref, i_ref, g_ref, lp_ref, md_ref):
    m = m_ref[...]
    mx = jnp.max(m, axis=0, keepdims=True)
    stot = jnp.sum(s_ref[...] * jnp.exp(m - mx), axis=0, keepdims=True)
    g = jnp.sum(g_ref[...], axis=0, keepdims=True)
    lp_ref[...] = g - (mx + jnp.log(stot))
    md_ref[...] = jnp.min(jnp.where(m == mx, i_ref[...], _I32_BIG),
                          axis=0, keepdims=True)


def kernel(logits, actions):
    act = actions.reshape(_B).astype(jnp.int32)
    lt = logits.T
    g_out, m_out, s_out, i_out = _sc_part(lt, act)
    lp, md = pl.pallas_call(
        _tc_merge,
        out_shape=(jax.ShapeDtypeStruct((1, _B), jnp.float32),
                   jax.ShapeDtypeStruct((1, _B), jnp.int32)),
    )(m_out.reshape(_NW, _B), s_out.reshape(_NW, _B),
      i_out.reshape(_NW, _B), g_out.reshape(_NW, _B))
    return lp.reshape(_B, 1), md.reshape(_B, 1)


# speedup vs baseline: 3.4582x; 1.1713x over previous
import functools

import jax
import jax.numpy as jnp
import numpy as np
from jax import lax
from jax.experimental import pallas as pl
from jax.experimental.pallas import tpu as pltpu
from jax.experimental.pallas import tpu_sc as plsc

_B = 128
_V = 100000
_L = 16
_NW = 32
_NBG = _B // _L
_VW = 3128
_W = 184
_TWR = _V - 31 * _VW - 16 * _W
_U1 = 2
_UT = 2

_NEG_INF = float("-inf")
_I32_BIG = np.int32(2**31 - 1)


def _sc_body(lt, act, g_out, m_out, s_out, i_out,
             buf_a, buf_b, buf_t, act_v, gst, mst, sst, ist,
             sem_a, sem_b, sem_t):
    w = lax.axis_index("s") * 2 + lax.axis_index("c")
    vb = w * _VW
    lane = lax.iota(jnp.int32, _L)
    neg = jnp.full((_L,), _NEG_INF, jnp.float32)
    zf = jnp.zeros((_L,), jnp.float32)
    zi = jnp.zeros((_L,), jnp.int32)

    pltpu.sync_copy(act, act_v)
    for bg in range(_NBG):
        mst[bg, :] = neg
        ist[bg, :] = zi
        sst[bg, :] = zf
        gst[bg, :] = zf

    def dma(c, buf, sem):
        pltpu.async_copy(lt.at[pl.ds(vb + c * _W, _W), :], buf, sem)

    def wait(buf, sem):
        pltpu.make_async_copy(lt.at[pl.ds(0, _W), :], buf, sem).wait()

    def process(buf, off, nv, un):
        m0 = [mst[bg, :] for bg in range(_NBG)]

        def p1(i, car):
            st = list(car)
            for u in range(un):
                o = i * un + u
                for bg in range(_NBG):
                    x = buf[o, pl.ds(bg * _L, _L)]
                    nm = jnp.maximum(st[bg], x)
                    st[_NBG + bg] = jnp.where(
                        x > st[bg], zi + (off + o), st[_NBG + bg])
                    st[bg] = nm
            return tuple(st)

        car = lax.fori_loop(
            0, nv // un, p1,
            tuple(m0) + tuple(ist[bg, :] for bg in range(_NBG)))
        m1 = car[:_NBG]
        for bg in range(_NBG):
            mst[bg, :] = m1[bg]
            ist[bg, :] = car[_NBG + bg]

        s0 = [jnp.where(m0[bg] > _NEG_INF,
                        sst[bg, :] * jnp.exp(m0[bg] - m1[bg]), 0.0)
              for bg in range(_NBG)]

        def p2(i, car):
            st = list(car)
            for u in range(un):
                o = i * un + u
                for bg in range(_NBG):
                    x = buf[o, pl.ds(bg * _L, _L)]
                    st[bg] = st[bg] + jnp.exp(x - m1[bg])
            return tuple(st)

        s2 = lax.fori_loop(0, nv // un, p2, tuple(s0))
        for bg in range(_NBG):
            sst[bg, :] = s2[bg]

        for bg in range(_NBG):
            av = act_v[pl.ds(bg * _L, _L)]
            loc = av - off
            inb = (loc >= 0) & (loc < nv)
            gath = plsc.load_gather(
                buf, [jnp.clip(loc, 0, nv - 1), bg * _L + lane])
            gst[bg, :] = jnp.where(inb, gath, gst[bg, :])

    dma(0, buf_a, sem_a)
    dma(1, buf_b, sem_b)

    @pl.loop(0, 16, step=2)
    def main_loop(c):
        wait(buf_a, sem_a)
        process(buf_a, vb + c * _W, _W, _U1)

        @pl.when(c < 14)
        def _():
            dma(c + 2, buf_a, sem_a)

        @pl.when(c == 14)
        def _():
            @pl.when(w < 31)
            def _():
                dma(16, buf_a, sem_a)

            @pl.when(w == 31)
            def _():
                pltpu.async_copy(
                    lt.at[pl.ds(vb + 16 * _W, _TWR), :], buf_t, sem_t)

        wait(buf_b, sem_b)
        process(buf_b, vb + (c + 1) * _W, _W, _U1)

        @pl.when(c < 13)
        def _():
            dma(c + 3, buf_b, sem_b)

    @pl.when(w < 31)
    def _():
        wait(buf_a, sem_a)
        process(buf_a, vb + 16 * _W, _W, _U1)

    @pl.when(w == 31)
    def _():
        pltpu.make_async_copy(
            lt.at[pl.ds(0, _TWR), :], buf_t, sem_t).wait()
        process(buf_t, vb + 16 * _W, _TWR, _UT)

    pltpu.sync_copy(gst, g_out.at[w])
    pltpu.sync_copy(mst, m_out.at[w])
    pltpu.sync_copy(sst, s_out.at[w])
    pltpu.sync_copy(ist, i_out.at[w])


_sc_part = functools.partial(
    pl.kernel,
    out_type=(
        jax.ShapeDtypeStruct((_NW, _NBG, _L), jnp.float32),
        jax.ShapeDtypeStruct((_NW, _NBG, _L), jnp.float32),
        jax.ShapeDtypeStruct((_NW, _NBG, _L), jnp.float32),
        jax.ShapeDtypeStruct((_NW, _NBG, _L), jnp.int32),
    ),
    mesh=plsc.VectorSubcoreMesh(
        core_axis_name="c", subcore_axis_name="s",
        num_cores=2, num_subcores=16),
    compiler_params=pltpu.CompilerParams(needs_layout_passes=False),
    scratch_types=[
        pltpu.VMEM((_W, _B), jnp.float32),
        pltpu.VMEM((_W, _B), jnp.float32),
        pltpu.VMEM((_TWR, _B), jnp.float32),
        pltpu.VMEM((_B,), jnp.int32),
        pltpu.VMEM((_NBG, _L), jnp.float32),
        pltpu.VMEM((_NBG, _L), jnp.float32),
        pltpu.VMEM((_NBG, _L), jnp.float32),
        pltpu.VMEM((_NBG, _L), jnp.int32),
        pltpu.SemaphoreType.DMA,
        pltpu.SemaphoreType.DMA,
        pltpu.SemaphoreType.DMA,
    ],
)(_sc_body)


def _tc_merge(m_ref, s_ref, i_ref, g_ref, lp_ref, md_ref):
    m = m_ref[...]
    mx = jnp.max(m, axis=0, keepdims=True)
    stot = jnp.sum(s_ref[...] * jnp.exp(m - mx), axis=0, keepdims=True)
    g = jnp.sum(g_ref[...], axis=0, keepdims=True)
    lp_ref[...] = g - (mx + jnp.log(stot))
    md_ref[...] = jnp.min(jnp.where(m == mx, i_ref[...], _I32_BIG),
                          axis=0, keepdims=True)


def kernel(logits, actions):
    act = actions.reshape(_B).astype(jnp.int32)
    lt = logits.T
    g_out, m_out, s_out, i_out = _sc_part(lt, act)
    lp, md = pl.pallas_call(
        _tc_merge,
        out_shape=(jax.ShapeDtypeStruct((1, _B), jnp.float32),
                   jax.ShapeDtypeStruct((1, _B), jnp.int32)),
    )(m_out.reshape(_NW, _B), s_out.reshape(_NW, _B),
      i_out.reshape(_NW, _B), g_out.reshape(_NW, _B))
    return lp.reshape(_B, 1), md.reshape(_B, 1)


# speedup vs baseline: 3.8411x; 1.1107x over previous
import functools

import jax
import jax.numpy as jnp
import numpy as np
from jax import lax
from jax.experimental import pallas as pl
from jax.experimental.pallas import tpu as pltpu
from jax.experimental.pallas import tpu_sc as plsc

_B = 128
_V = 100000
_L = 16
_NW = 32
_NBG = _B // _L
_VW = 3128
_W = 184
_TWR = _V - 31 * _VW - 16 * _W
_U1 = 2
_UT = 2

_NEG_INF = float("-inf")
_I32_BIG = np.int32(2**31 - 1)


def _sc_body(lt, act, g_out, m_out, s_out, i_out,
             buf_a, buf_b, buf_t, act_v, gst, mst, sst, ist,
             sem_a, sem_b, sem_t):
    w = lax.axis_index("s") * 2 + lax.axis_index("c")
    vb = w * _VW
    lane = lax.iota(jnp.int32, _L)
    neg = jnp.full((_L,), _NEG_INF, jnp.float32)
    zf = jnp.zeros((_L,), jnp.float32)
    zi = jnp.zeros((_L,), jnp.int32)

    pltpu.sync_copy(act, act_v)
    for bg in range(_NBG):
        mst[pl.ds(bg * _L, _L)] = neg
        ist[pl.ds(bg * _L, _L)] = zi
        sst[pl.ds(bg * _L, _L)] = zf
        gst[pl.ds(bg * _L, _L)] = zf

    def dma(c, buf, sem):
        pltpu.async_copy(lt.at[pl.ds(vb + c * _W, _W), :], buf, sem)

    def wait(buf, sem):
        pltpu.make_async_copy(lt.at[pl.ds(0, _W), :], buf, sem).wait()

    def process(buf, off, nv, un):
        m0 = [mst[pl.ds(bg * _L, _L)] for bg in range(_NBG)]

        def p1(i, car):
            st = list(car)
            for u in range(un):
                o = i * un + u
                for bg in range(_NBG):
                    x = buf[o, pl.ds(bg * _L, _L)]
                    nm = jnp.maximum(st[bg], x)
                    st[_NBG + bg] = jnp.where(
                        x > st[bg], zi + (off + o), st[_NBG + bg])
                    st[bg] = nm
            return tuple(st)

        car = lax.fori_loop(
            0, nv // un, p1,
            tuple(m0) + tuple(ist[pl.ds(bg * _L, _L)] for bg in range(_NBG)))
        m1 = car[:_NBG]
        for bg in range(_NBG):
            mst[pl.ds(bg * _L, _L)] = m1[bg]
            ist[pl.ds(bg * _L, _L)] = car[_NBG + bg]

        s0 = [jnp.where(m0[bg] > _NEG_INF,
                        sst[pl.ds(bg * _L, _L)] * jnp.exp(m0[bg] - m1[bg]),
                        0.0)
              for bg in range(_NBG)]

        def p2(i, car):
            st = list(car)
            for u in range(un):
                o = i * un + u
                for bg in range(_NBG):
                    x = buf[o, pl.ds(bg * _L, _L)]
                    st[bg] = st[bg] + jnp.exp(x - m1[bg])
            return tuple(st)

        s2 = lax.fori_loop(0, nv // un, p2, tuple(s0))
        for bg in range(_NBG):
            sst[pl.ds(bg * _L, _L)] = s2[bg]

        for bg in range(_NBG):
            av = act_v[pl.ds(bg * _L, _L)]
            loc = av - off
            inb = (loc >= 0) & (loc < nv)
            gath = plsc.load_gather(
                buf, [jnp.clip(loc, 0, nv - 1), bg * _L + lane])
            gst[pl.ds(bg * _L, _L)] = jnp.where(
                inb, gath, gst[pl.ds(bg * _L, _L)])

    dma(0, buf_a, sem_a)
    dma(1, buf_b, sem_b)

    @pl.loop(0, 16, step=2)
    def main_loop(c):
        wait(buf_a, sem_a)
        process(buf_a, vb + c * _W, _W, _U1)

        @pl.when(c < 14)
        def _():
            dma(c + 2, buf_a, sem_a)

        @pl.when(c == 14)
        def _():
            @pl.when(w < 31)
            def _():
                dma(16, buf_a, sem_a)

            @pl.when(w == 31)
            def _():
                pltpu.async_copy(
                    lt.at[pl.ds(vb + 16 * _W, _TWR), :], buf_t, sem_t)

        wait(buf_b, sem_b)
        process(buf_b, vb + (c + 1) * _W, _W, _U1)

        @pl.when(c < 13)
        def _():
            dma(c + 3, buf_b, sem_b)

    @pl.when(w < 31)
    def _():
        wait(buf_a, sem_a)
        process(buf_a, vb + 16 * _W, _W, _U1)

    @pl.when(w == 31)
    def _():
        pltpu.make_async_copy(
            lt.at[pl.ds(0, _TWR), :], buf_t, sem_t).wait()
        process(buf_t, vb + 16 * _W, _TWR, _UT)

    pltpu.sync_copy(gst, g_out.at[w])
    pltpu.sync_copy(mst, m_out.at[w])
    pltpu.sync_copy(sst, s_out.at[w])
    pltpu.sync_copy(ist, i_out.at[w])


_sc_part = functools.partial(
    pl.kernel,
    out_type=(
        jax.ShapeDtypeStruct((_NW, _B), jnp.float32),
        jax.ShapeDtypeStruct((_NW, _B), jnp.float32),
        jax.ShapeDtypeStruct((_NW, _B), jnp.float32),
        jax.ShapeDtypeStruct((_NW, _B), jnp.int32),
    ),
    mesh=plsc.VectorSubcoreMesh(
        core_axis_name="c", subcore_axis_name="s",
        num_cores=2, num_subcores=16),
    compiler_params=pltpu.CompilerParams(needs_layout_passes=False),
    scratch_types=[
        pltpu.VMEM((_W, _B), jnp.float32),
        pltpu.VMEM((_W, _B), jnp.float32),
        pltpu.VMEM((_TWR, _B), jnp.float32),
        pltpu.VMEM((_B,), jnp.int32),
        pltpu.VMEM((_B,), jnp.float32),
        pltpu.VMEM((_B,), jnp.float32),
        pltpu.VMEM((_B,), jnp.float32),
        pltpu.VMEM((_B,), jnp.int32),
        pltpu.SemaphoreType.DMA,
        pltpu.SemaphoreType.DMA,
        pltpu.SemaphoreType.DMA,
    ],
)(_sc_body)


def _tc_merge(m_ref, s_ref, i_ref, g_ref, lp_ref, md_ref):
    m = m_ref[...]
    mx = jnp.max(m, axis=0, keepdims=True)
    stot = jnp.sum(s_ref[...] * jnp.exp(m - mx), axis=0, keepdims=True)
    g = jnp.sum(g_ref[...], axis=0, keepdims=True)
    lp_ref[...] = g - (mx + jnp.log(stot))
    md_ref[...] = jnp.min(jnp.where(m == mx, i_ref[...], _I32_BIG),
                          axis=0, keepdims=True)


def kernel(logits, actions):
    act = actions.reshape(_B).astype(jnp.int32)
    lt = logits.T
    g_out, m_out, s_out, i_out = _sc_part(lt, act)
    lp, md = pl.pallas_call(
        _tc_merge,
        out_shape=(jax.ShapeDtypeStruct((1, _B), jnp.float32),
                   jax.ShapeDtypeStruct((1, _B), jnp.int32)),
    )(m_out, s_out, i_out, g_out)
    return lp.reshape(_B, 1), md.reshape(_B, 1)
